# baseline (speedup 1.0000x reference)
import functools

import jax
import jax.numpy as jnp
from jax import lax
from jax.experimental import pallas as pl
from jax.experimental.pallas import tpu as pltpu
from jax.experimental.pallas import tpu_sc as plsc

_BS = 4096
_NS = 16
_NE = 8
_D = 64
_L = 16
_DC = _D // _L
_ROW_E = _NS * _NE * _D
_ROW_M = _NE * _NS
_ROW_N = _NS * _D


def _sc_body(edge_hbm, maskt_hbm, ent_hbm, self_hbm, nv_hbm, sv_hbm,
             ebuf, mbuf, entbuf, selfbuf, nvbuf, svbuf):
    info = plsc.get_sparse_core_info()
    nc = info.num_cores
    wid = lax.axis_index("s") * nc + lax.axis_index("c")
    rows_per_w = _BS // (nc * info.num_subcores)
    base = wid * rows_per_w

    def row_body(i, carry):
        row = base + i
        pltpu.sync_copy(edge_hbm.at[row], ebuf)
        pltpu.sync_copy(maskt_hbm.at[row], mbuf)
        pltpu.sync_copy(ent_hbm.at[row], entbuf)
        pltpu.sync_copy(self_hbm.at[row], selfbuf)

        m_vecs = [mbuf[pl.ds(e * _L, _L)] for e in range(_NE)]
        cnt = m_vecs[0]
        for e in range(1, _NE):
            cnt = cnt + m_vecs[e]
        denom = jnp.where(cnt == 0.0, 1.0, cnt)
        inv = 1.0 / denom
        w_vecs = [m_vecs[e] * inv for e in range(_NE)]

        sv_acc = [jnp.zeros((_L,), jnp.float32) for _ in range(_DC)]
        for s in range(_NS):
            ws = [w_vecs[e][s] for e in range(_NE)]
            for dc in range(_DC):
                off = s * (_NE * _D) + dc * _L
                acc = ws[0] * ebuf[pl.ds(off, _L)]
                for e in range(1, _NE):
                    acc = acc + ws[e] * ebuf[pl.ds(off + e * _D, _L)]
                nv = entbuf[pl.ds(s * _D + dc * _L, _L)] + acc
                nvbuf[pl.ds(s * _D + dc * _L, _L)] = nv
                sv_acc[dc] = sv_acc[dc] + nv

        for dc in range(_DC):
            svbuf[pl.ds(dc * _L, _L)] = (
                selfbuf[pl.ds(dc * _L, _L)] + sv_acc[dc] * (1.0 / _NS))

        pltpu.sync_copy(nvbuf, nv_hbm.at[row])
        pltpu.sync_copy(svbuf, sv_hbm.at[row])
        return carry

    lax.fori_loop(0, rows_per_w, row_body, 0)


@jax.jit
def _run(edge, maskt, ent, selfv):
    mesh = plsc.VectorSubcoreMesh(core_axis_name="c", subcore_axis_name="s")
    body = functools.partial(
        pl.kernel,
        mesh=mesh,
        out_type=(
            jax.ShapeDtypeStruct((_BS, _ROW_N), jnp.float32),
            jax.ShapeDtypeStruct((_BS, _D), jnp.float32),
        ),
        scratch_types=[
            pltpu.VMEM((_ROW_E,), jnp.float32),
            pltpu.VMEM((_ROW_M,), jnp.float32),
            pltpu.VMEM((_ROW_N,), jnp.float32),
            pltpu.VMEM((_D,), jnp.float32),
            pltpu.VMEM((_ROW_N,), jnp.float32),
            pltpu.VMEM((_D,), jnp.float32),
        ],
    )(_sc_body)
    return body(edge, maskt, ent, selfv)


def kernel(self_vectors, neighbor_entity_vectors, neighbor_edge_vectors, masks):
    bs = self_vectors.shape[0]
    edge = neighbor_edge_vectors.reshape(bs, _ROW_E)
    maskt = jnp.swapaxes(masks.reshape(bs, _NS, _NE), 1, 2).reshape(bs, _ROW_M)
    ent = neighbor_entity_vectors.reshape(bs, _ROW_N)
    selfv = self_vectors.reshape(bs, _D)
    nv, sv = _run(edge, maskt, ent, selfv)
    return (sv.reshape(bs, 1, _D), nv.reshape(bs, 1, _NS, _D))


# speedup vs baseline: 1.6508x; 1.6508x over previous
import functools

import jax
import jax.numpy as jnp
from jax import lax
from jax.experimental import pallas as pl
from jax.experimental.pallas import tpu as pltpu
from jax.experimental.pallas import tpu_sc as plsc

_BS = 4096
_NS = 16
_NE = 8
_D = 64
_L = 16
_DC = _D // _L
_ROW_E = _NS * _NE * _D
_ROW_M = _NE * _NS
_ROW_N = _NS * _D
_ROW_A = _ROW_M + _ROW_N + _D
_ROW_O = _ROW_N + _D
_A_ENT = _ROW_M
_A_SELF = _ROW_M + _ROW_N


def _compute_row(ebuf, abuf, obuf):
    m_vecs = [abuf[pl.ds(e * _L, _L)] for e in range(_NE)]
    cnt = m_vecs[0]
    for e in range(1, _NE):
        cnt = cnt + m_vecs[e]
    inv = 1.0 / jnp.where(cnt == 0.0, 1.0, cnt)
    w_vecs = [m_vecs[e] * inv for e in range(_NE)]

    sv_acc = [None] * _DC
    for s in range(_NS):
        ws = [w_vecs[e][s] for e in range(_NE)]
        for dc in range(_DC):
            off = s * (_NE * _D) + dc * _L
            acc = ws[0] * ebuf[pl.ds(off, _L)]
            for e in range(1, _NE):
                acc = acc + ws[e] * ebuf[pl.ds(off + e * _D, _L)]
            nv = abuf[pl.ds(_A_ENT + s * _D + dc * _L, _L)] + acc
            obuf[pl.ds(s * _D + dc * _L, _L)] = nv
            sv_acc[dc] = nv if s == 0 else sv_acc[dc] + nv

    for dc in range(_DC):
        obuf[pl.ds(_ROW_N + dc * _L, _L)] = (
            abuf[pl.ds(_A_SELF + dc * _L, _L)] + sv_acc[dc] * (1.0 / _NS))


def _sc_body(edge_hbm, aux_hbm, out_hbm,
             ebuf0, ebuf1, abuf0, abuf1, obuf0, obuf1,
             si0, si1, so0, so1):
    info = plsc.get_sparse_core_info()
    nc = info.num_cores
    wid = lax.axis_index("s") * nc + lax.axis_index("c")
    nrows = _BS // (nc * info.num_subcores)
    base = wid * nrows
    npairs = nrows // 2

    def start_in(row, ebuf, abuf, si):
        pltpu.async_copy(edge_hbm.at[row], ebuf, si)
        pltpu.async_copy(aux_hbm.at[row], abuf, si)

    def wait_in(row, ebuf, abuf, si):
        pltpu.make_async_copy(edge_hbm.at[row], ebuf, si).wait()
        pltpu.make_async_copy(aux_hbm.at[row], abuf, si).wait()

    start_in(base, ebuf0, abuf0, si0)

    def pair_body(g, carry):
        r0 = base + 2 * g
        r1 = r0 + 1

        start_in(r1, ebuf1, abuf1, si1)
        wait_in(r0, ebuf0, abuf0, si0)

        @pl.when(g >= 1)
        def _():
            pltpu.make_async_copy(obuf0, out_hbm.at[r0 - 2], so0).wait()

        _compute_row(ebuf0, abuf0, obuf0)
        pltpu.async_copy(obuf0, out_hbm.at[r0], so0)

        @pl.when(g < npairs - 1)
        def _():
            start_in(r1 + 1, ebuf0, abuf0, si0)

        wait_in(r1, ebuf1, abuf1, si1)

        @pl.when(g >= 1)
        def _():
            pltpu.make_async_copy(obuf1, out_hbm.at[r1 - 2], so1).wait()

        _compute_row(ebuf1, abuf1, obuf1)
        pltpu.async_copy(obuf1, out_hbm.at[r1], so1)
        return carry

    lax.fori_loop(0, npairs, pair_body, 0)

    pltpu.make_async_copy(obuf0, out_hbm.at[base + nrows - 2], so0).wait()
    pltpu.make_async_copy(obuf1, out_hbm.at[base + nrows - 1], so1).wait()


@jax.jit
def _run(edge, aux):
    mesh = plsc.VectorSubcoreMesh(core_axis_name="c", subcore_axis_name="s")
    body = functools.partial(
        pl.kernel,
        mesh=mesh,
        out_type=jax.ShapeDtypeStruct((_BS, _ROW_O), jnp.float32),
        scratch_types=[
            pltpu.VMEM((_ROW_E,), jnp.float32),
            pltpu.VMEM((_ROW_E,), jnp.float32),
            pltpu.VMEM((_ROW_A,), jnp.float32),
            pltpu.VMEM((_ROW_A,), jnp.float32),
            pltpu.VMEM((_ROW_O,), jnp.float32),
            pltpu.VMEM((_ROW_O,), jnp.float32),
            pltpu.SemaphoreType.DMA,
            pltpu.SemaphoreType.DMA,
            pltpu.SemaphoreType.DMA,
            pltpu.SemaphoreType.DMA,
        ],
    )(_sc_body)
    return body(edge, aux)


def kernel(self_vectors, neighbor_entity_vectors, neighbor_edge_vectors, masks):
    bs = self_vectors.shape[0]
    edge = neighbor_edge_vectors.reshape(bs, _ROW_E)
    maskt = jnp.swapaxes(masks.reshape(bs, _NS, _NE), 1, 2).reshape(bs, _ROW_M)
    aux = jnp.concatenate(
        [maskt,
         neighbor_entity_vectors.reshape(bs, _ROW_N),
         self_vectors.reshape(bs, _D)], axis=1)
    out = _run(edge, aux)
    sv = out[:, _ROW_N:]
    nv = out[:, :_ROW_N]
    return (sv.reshape(bs, 1, _D), nv.reshape(bs, 1, _NS, _D))
